# baseline (speedup 1.0000x reference)
import functools

import jax
import jax.numpy as jnp
from jax.experimental import pallas as pl
from jax.experimental.pallas import tpu as pltpu

NUM_CLUSTERS = 32
WINDOW_SIZE = 128
COMMITMENT = 1e-4

INTERPRET = False


def _dists_body(q_ref, k_ref, m_ref, dq_ref, dk_ref, aux_ref):
    m = m_ref[0]
    msq = jnp.sum(m * m, axis=1)

    def stats(x):
        nrm = jnp.sqrt(jnp.sum(x * x, axis=1, keepdims=True))
        xn = x / jnp.maximum(nrm, 1e-12)
        d = jax.lax.dot_general(m, xn, (((1,), (1,)), ((), ())),
                                preferred_element_type=jnp.float32)
        s = jnp.sum(xn * xn, axis=1)
        dmax = jnp.max(d, axis=0)
        amax = jnp.argmax(d, axis=0)
        sel = jax.lax.broadcasted_iota(jnp.int32, d.shape, 0) == amax[None, :]
        msqsel = jnp.sum(jnp.where(sel, msq[:, None], 0.0), axis=0)
        part = jnp.sum(s - 2.0 * dmax + msqsel)
        return d, part

    dq, pq = stats(q_ref[0, 0])
    dk, pk = stats(k_ref[0, 0])
    dq_ref[0, 0] = dq
    dk_ref[0, 0] = dk
    r = jax.lax.broadcasted_iota(jnp.int32, (8, 128), 0)
    c = jax.lax.broadcasted_iota(jnp.int32, (8, 128), 1)
    aux_ref[0, 0] = jnp.where((r == 0) & (c == 0), pq + pk, 0.0)


def _dists_call(q, k, means):
    b, h, t, d = q.shape
    nc = means.shape[1]
    grid = (b, h)
    return pl.pallas_call(
        _dists_body,
        grid=grid,
        in_specs=[
            pl.BlockSpec((1, 1, t, d), lambda i, j: (i, j, 0, 0)),
            pl.BlockSpec((1, 1, t, d), lambda i, j: (i, j, 0, 0)),
            pl.BlockSpec((1, nc, d), lambda i, j: (j, 0, 0)),
        ],
        out_specs=[
            pl.BlockSpec((1, 1, nc, t), lambda i, j: (i, j, 0, 0)),
            pl.BlockSpec((1, 1, nc, t), lambda i, j: (i, j, 0, 0)),
            pl.BlockSpec((1, 1, 8, 128), lambda i, j: (i, j, 0, 0)),
        ],
        out_shape=[
            jax.ShapeDtypeStruct((b, h, nc, t), jnp.float32),
            jax.ShapeDtypeStruct((b, h, nc, t), jnp.float32),
            jax.ShapeDtypeStruct((b, h, 8, 128), jnp.float32),
        ],
        interpret=INTERPRET,
    )(q, k, means)


def _attn_body(qg_ref, kg_ref, vg_ref, o_ref, *, nc, wsz, scale):
    for c in range(nc):
        sl = slice(c * wsz, (c + 1) * wsz)
        qc = qg_ref[0, sl, :]
        kc = kg_ref[0, sl, :]
        vc = vg_ref[0, sl, :]
        dots = jax.lax.dot_general(qc, kc, (((1,), (1,)), ((), ())),
                                   preferred_element_type=jnp.float32) * scale
        mx = jnp.max(dots, axis=1, keepdims=True)
        e = jnp.exp(dots - mx)
        p = e / jnp.sum(e, axis=1, keepdims=True)
        o_ref[0, sl, :] = jnp.dot(p, vc, preferred_element_type=jnp.float32)


def _attn_call(qg, kg, vg):
    bh, n, d = qg.shape
    nc = NUM_CLUSTERS
    wsz = n // nc
    body = functools.partial(_attn_body, nc=nc, wsz=wsz, scale=d ** -0.5)
    return pl.pallas_call(
        body,
        grid=(bh,),
        in_specs=[pl.BlockSpec((1, n, d), lambda i: (i, 0, 0))] * 3,
        out_specs=pl.BlockSpec((1, n, d), lambda i: (i, 0, 0)),
        out_shape=jax.ShapeDtypeStruct((bh, n, d), jnp.float32),
        interpret=INTERPRET,
    )(qg, kg, vg)


def kernel(q, k, v, means):
    b, h, t, d = q.shape
    nc = NUM_CLUSTERS
    wsz = min(WINDOW_SIZE, t)

    dq, dk, aux_parts = _dists_call(q, k, means)
    aux_loss = jnp.sum(aux_parts) * (COMMITMENT / (b * h * 2 * t * d))

    _, idx_q = jax.lax.top_k(dq, wsz)
    _, idx_k = jax.lax.top_k(dk, wsz)

    iq = idx_q.reshape(b, h, nc * wsz)
    ik = idx_k.reshape(b, h, nc * wsz)
    qg = jnp.take_along_axis(q, iq[..., None], axis=2)
    kg = jnp.take_along_axis(k, ik[..., None], axis=2)
    vg = jnp.take_along_axis(v, ik[..., None], axis=2)

    so = _attn_call(qg.reshape(b * h, nc * wsz, d),
                    kg.reshape(b * h, nc * wsz, d),
                    vg.reshape(b * h, nc * wsz, d)).reshape(b, h, nc * wsz, d)

    def _one(t_bh, idx_bh):
        numer = jnp.zeros((t, d), jnp.float32).at[idx_bh].add(t_bh)
        count = jnp.zeros((t,), jnp.float32).at[idx_bh].add(1.0)
        return numer / (count[:, None] + 1e-5)

    out = jax.vmap(jax.vmap(_one))(so, iq)
    return out, aux_loss


# speedup vs baseline: 2.0102x; 2.0102x over previous
import functools

import jax
import jax.numpy as jnp
from jax import lax
from jax.experimental import pallas as pl
from jax.experimental.pallas import tpu as pltpu
from jax.experimental.pallas import tpu_sc as plsc

NUM_CLUSTERS = 32
WINDOW_SIZE = 128
COMMITMENT = 1e-4

INTERPRET = False

_MININT = -2147483648


def _select_threshold(dmat, wsz):
    s = lax.bitcast_convert_type(dmat, jnp.int32)
    key = jnp.where(s >= 0, s, s ^ jnp.int32(0x7FFFFFFF))

    def bit_body(i, p):
        c = p | lax.shift_left(jnp.int32(1), 31 - i)
        cs = c ^ _MININT
        cnt = jnp.sum((key >= cs).astype(jnp.int32), axis=1, keepdims=True)
        return jnp.where(cnt >= wsz, c, p)

    p = lax.fori_loop(0, 32, bit_body, jnp.zeros((dmat.shape[0], 1), jnp.int32))
    thr = p ^ _MININT
    cnt_gt = jnp.sum((key > thr).astype(jnp.int32), axis=1, keepdims=True)
    return thr, wsz - cnt_gt


def _dists_body(q_ref, k_ref, m_ref, dq_ref, dk_ref, aux_ref,
                thrq_ref, budq_ref, thrk_ref, budk_ref, *, wsz):
    m = m_ref[0]
    msq = jnp.sum(m * m, axis=1)

    def stats(x):
        nrm = jnp.sqrt(jnp.sum(x * x, axis=1, keepdims=True))
        xn = x / jnp.maximum(nrm, 1e-12)
        d = lax.dot_general(m, xn, (((1,), (1,)), ((), ())),
                            preferred_element_type=jnp.float32)
        s = jnp.sum(xn * xn, axis=1)
        dmax = jnp.max(d, axis=0)
        amax = jnp.argmax(d, axis=0)
        sel = lax.broadcasted_iota(jnp.int32, d.shape, 0) == amax[None, :]
        msqsel = jnp.sum(jnp.where(sel, msq[:, None], 0.0), axis=0)
        part = jnp.sum(s - 2.0 * dmax + msqsel)
        return d, part

    dq, pq = stats(q_ref[0, 0])
    dk, pk = stats(k_ref[0, 0])
    dq_ref[0, 0] = dq
    dk_ref[0, 0] = dk
    r = lax.broadcasted_iota(jnp.int32, (8, 128), 0)
    c = lax.broadcasted_iota(jnp.int32, (8, 128), 1)
    aux_ref[0, 0] = jnp.where((r == 0) & (c == 0), pq + pk, 0.0)

    thrq, budq = _select_threshold(dq, wsz)
    thrk, budk = _select_threshold(dk, wsz)
    thrq_ref[0, 0] = thrq
    budq_ref[0, 0] = budq
    thrk_ref[0, 0] = thrk
    budk_ref[0, 0] = budk


def _dists_call(q, k, means):
    b, h, t, d = q.shape
    nc = means.shape[1]
    wsz = min(WINDOW_SIZE, t)
    grid = (b, h)
    i32 = jnp.int32
    return pl.pallas_call(
        functools.partial(_dists_body, wsz=wsz),
        grid=grid,
        in_specs=[
            pl.BlockSpec((1, 1, t, d), lambda i, j: (i, j, 0, 0)),
            pl.BlockSpec((1, 1, t, d), lambda i, j: (i, j, 0, 0)),
            pl.BlockSpec((1, nc, d), lambda i, j: (j, 0, 0)),
        ],
        out_specs=[
            pl.BlockSpec((1, 1, nc, t), lambda i, j: (i, j, 0, 0)),
            pl.BlockSpec((1, 1, nc, t), lambda i, j: (i, j, 0, 0)),
            pl.BlockSpec((1, 1, 8, 128), lambda i, j: (i, j, 0, 0)),
            pl.BlockSpec((1, 1, nc, 1), lambda i, j: (i, j, 0, 0)),
            pl.BlockSpec((1, 1, nc, 1), lambda i, j: (i, j, 0, 0)),
            pl.BlockSpec((1, 1, nc, 1), lambda i, j: (i, j, 0, 0)),
            pl.BlockSpec((1, 1, nc, 1), lambda i, j: (i, j, 0, 0)),
        ],
        out_shape=[
            jax.ShapeDtypeStruct((b, h, nc, t), jnp.float32),
            jax.ShapeDtypeStruct((b, h, nc, t), jnp.float32),
            jax.ShapeDtypeStruct((b, h, 8, 128), jnp.float32),
            jax.ShapeDtypeStruct((b, h, nc, 1), i32),
            jax.ShapeDtypeStruct((b, h, nc, 1), i32),
            jax.ShapeDtypeStruct((b, h, nc, 1), i32),
            jax.ShapeDtypeStruct((b, h, nc, 1), i32),
        ],
        interpret=INTERPRET,
    )(q, k, means)


def _lane_extract(vec_ref, i):
    base = (i // 16) * 16
    v = vec_ref[pl.ds(base, 16)]
    lane = lax.broadcasted_iota(jnp.int32, (16,), 0)
    return jnp.sum(jnp.where(lane == (i - base), v, 0))


def _compact_row(rowv, thr, bud, idxv, t):
    def chunk(j, carry):
        eqc, sc = carry
        vals = rowv[pl.ds(j * 16, 16)]
        s = plsc.bitcast(vals, jnp.int32)
        key = jnp.where(s >= 0, s, s ^ jnp.int32(0x7FFFFFFF))
        gt = key > thr
        eq = key == thr
        eqrank = plsc.cumsum(eq.astype(jnp.int32)) + eqc
        sel = jnp.logical_or(gt, jnp.logical_and(eq, eqrank <= bud))
        seli = sel.astype(jnp.int32)
        pos = plsc.cumsum(seli) - 1 + sc
        pos = jnp.where(sel, pos, 0)
        col = lax.broadcasted_iota(jnp.int32, (16,), 0) + j * 16
        plsc.store_scatter(idxv, [pos], col, mask=sel)
        return eqc + jnp.sum(eq.astype(jnp.int32)), sc + jnp.sum(seli)

    lax.fori_loop(0, t // 16, chunk, (jnp.int32(0), jnp.int32(0)))


def _route_call(q, k, v, dq, dk, thrq, budq, thrk, budk):
    b, h, t, d = q.shape
    nc = NUM_CLUSTERS
    wsz = min(WINDOW_SIZE, t)
    info = plsc.get_sparse_core_info()
    n_tec = info.num_cores * info.num_subcores
    per_q = -(-(b * h * nc) // n_tec)
    n_task = b * h * nc
    mesh = plsc.VectorSubcoreMesh(core_axis_name="c", subcore_axis_name="s")

    @functools.partial(
        pl.kernel,
        mesh=mesh,
        compiler_params=pltpu.CompilerParams(
            needs_layout_passes=False, use_tc_tiling_on_sc=False),
        out_type=[
            jax.ShapeDtypeStruct((b, h, nc * wsz), jnp.int32),
            jax.ShapeDtypeStruct((b, h, nc * wsz, d), jnp.float32),
            jax.ShapeDtypeStruct((b, h, nc * wsz, d), jnp.float32),
            jax.ShapeDtypeStruct((b, h, nc * wsz, d), jnp.float32),
        ],
        scratch_types=[
            pltpu.VMEM((t,), jnp.float32),
            pltpu.VMEM((wsz,), jnp.int32),
            pltpu.VMEM((wsz, d), jnp.float32),
            pltpu.VMEM((wsz, d), jnp.float32),
            pltpu.VMEM((nc,), jnp.int32),
            pltpu.VMEM((nc,), jnp.int32),
            pltpu.SemaphoreType.DMA,
        ],
    )
    def route(q_hbm, k_hbm, v_hbm, dq_hbm, dk_hbm, thrq_hbm, budq_hbm,
              thrk_hbm, budk_hbm, iq_hbm, qg_hbm, kg_hbm, vg_hbm,
              rowv, idxv, gb1, gb2, thrv, budv, sem):
        cid = lax.axis_index("c")
        sid = lax.axis_index("s")
        wid = sid * info.num_cores + cid

        def q_body(r, carry):
            task = wid * per_q + r

            @pl.when(task < n_task)
            def _():
                c_i = task % nc
                h_i = (task // nc) % h
                b_i = task // (nc * h)
                pltpu.sync_copy(dq_hbm.at[b_i, h_i, c_i], rowv)
                pltpu.sync_copy(thrq_hbm.at[b_i, h_i], thrv)
                pltpu.sync_copy(budq_hbm.at[b_i, h_i], budv)
                thr = _lane_extract(thrv, c_i)
                bud = _lane_extract(budv, c_i)
                _compact_row(rowv, thr, bud, idxv, t)
                pltpu.sync_copy(idxv, iq_hbm.at[b_i, h_i, pl.ds(c_i * wsz, wsz)])
                pltpu.async_copy(q_hbm.at[b_i, h_i].at[idxv], gb1, sem).wait()
                pltpu.sync_copy(gb1, qg_hbm.at[b_i, h_i, pl.ds(c_i * wsz, wsz)])

            return carry

        lax.fori_loop(0, per_q, q_body, 0)

        def k_body(r, carry):
            task = wid * per_q + r

            @pl.when(task < n_task)
            def _():
                c_i = task % nc
                h_i = (task // nc) % h
                b_i = task // (nc * h)
                pltpu.sync_copy(dk_hbm.at[b_i, h_i, c_i], rowv)
                pltpu.sync_copy(thrk_hbm.at[b_i, h_i], thrv)
                pltpu.sync_copy(budk_hbm.at[b_i, h_i], budv)
                thr = _lane_extract(thrv, c_i)
                bud = _lane_extract(budv, c_i)
                _compact_row(rowv, thr, bud, idxv, t)
                pltpu.async_copy(k_hbm.at[b_i, h_i].at[idxv], gb1, sem).wait()
                pltpu.async_copy(v_hbm.at[b_i, h_i].at[idxv], gb2, sem).wait()
                pltpu.sync_copy(gb1, kg_hbm.at[b_i, h_i, pl.ds(c_i * wsz, wsz)])
                pltpu.sync_copy(gb2, vg_hbm.at[b_i, h_i, pl.ds(c_i * wsz, wsz)])

            return carry

        lax.fori_loop(0, per_q, k_body, 0)

    return route(q, k, v, dq, dk, thrq, budq, thrk, budk)


def _attn_body(qg_ref, kg_ref, vg_ref, o_ref, *, nc, wsz, scale):
    for c in range(nc):
        sl = slice(c * wsz, (c + 1) * wsz)
        qc = qg_ref[0, sl, :]
        kc = kg_ref[0, sl, :]
        vc = vg_ref[0, sl, :]
        dots = lax.dot_general(qc, kc, (((1,), (1,)), ((), ())),
                               preferred_element_type=jnp.float32) * scale
        mx = jnp.max(dots, axis=1, keepdims=True)
        e = jnp.exp(dots - mx)
        p = e / jnp.sum(e, axis=1, keepdims=True)
        o_ref[0, sl, :] = jnp.dot(p, vc, preferred_element_type=jnp.float32)


def _attn_call(qg, kg, vg):
    bh, n, d = qg.shape
    nc = NUM_CLUSTERS
    wsz = n // nc
    body = functools.partial(_attn_body, nc=nc, wsz=wsz, scale=d ** -0.5)
    return pl.pallas_call(
        body,
        grid=(bh,),
        in_specs=[pl.BlockSpec((1, n, d), lambda i: (i, 0, 0))] * 3,
        out_specs=pl.BlockSpec((1, n, d), lambda i: (i, 0, 0)),
        out_shape=jax.ShapeDtypeStruct((bh, n, d), jnp.float32),
        interpret=INTERPRET,
    )(qg, kg, vg)


def kernel(q, k, v, means):
    b, h, t, d = q.shape
    nc = NUM_CLUSTERS
    wsz = min(WINDOW_SIZE, t)

    dq, dk, aux_parts, thrq, budq, thrk, budk = _dists_call(q, k, means)
    aux_loss = jnp.sum(aux_parts) * (COMMITMENT / (b * h * 2 * t * d))

    iq, qg, kg, vg = _route_call(
        q, k, v, dq, dk,
        thrq.reshape(b, h, nc), budq.reshape(b, h, nc),
        thrk.reshape(b, h, nc), budk.reshape(b, h, nc))

    so = _attn_call(qg.reshape(b * h, nc * wsz, d),
                    kg.reshape(b * h, nc * wsz, d),
                    vg.reshape(b * h, nc * wsz, d)).reshape(b, h, nc * wsz, d)

    def _one(t_bh, idx_bh):
        z = jnp.zeros((t, d), jnp.float32)
        numer = z.at[idx_bh].add(t_bh)
        denom = z.at[idx_bh].add(jnp.ones_like(t_bh))
        return numer / (denom + 1e-5)

    out = jax.vmap(jax.vmap(_one))(so, iq)
    return out, aux_loss
